# baseline (speedup 1.0000x reference)
import jax
import jax.numpy as jnp
from jax import lax
from jax.experimental import pallas as pl
from jax.experimental.pallas import tpu as pltpu
from jax.experimental.pallas import tpu_sc as plsc

BATCH = 16384
DIMS = 64
LANES = 16
VROW = 2 * DIMS

_info = plsc.get_sparse_core_info()
NC = _info.num_cores
NS = _info.num_subcores
NW = NC * NS

B_PER_W = BATCH // NW
CHUNK = 16
N_CHUNKS = B_PER_W // CHUNK


def _gmf_body(u_hbm, m_hbm, table_hbm, out_hbm,
              idx_u, idx_m, blk_u, blk_m, outb,
              sem_u0, sem_m0, sem_u1, sem_m1):
    wid = lax.axis_index("s") * NC + lax.axis_index("c")
    base = pl.multiple_of(wid * B_PER_W, B_PER_W)

    pltpu.sync_copy(u_hbm.at[pl.ds(base, B_PER_W)], idx_u)
    pltpu.sync_copy(m_hbm.at[pl.ds(base, B_PER_W)], idx_m)

    def issue(k, buf, sem_u, sem_m):
        def wave(c16, _):
            vu = idx_u[pl.ds(k * CHUNK + c16 * LANES, LANES)]
            vm = idx_m[pl.ds(k * CHUNK + c16 * LANES, LANES)]
            for l in range(LANES):
                c = c16 * LANES + l
                ru = pl.multiple_of(
                    lax.shift_left(lax.shift_right_logical(vu[l], 3), 3), 8)
                rm = pl.multiple_of(
                    lax.shift_left(lax.shift_right_logical(vm[l], 3), 3), 8)
                pltpu.async_copy(
                    table_hbm.at[pl.ds(ru, 8), :], blk_u.at[buf, c], sem_u)
                pltpu.async_copy(
                    table_hbm.at[pl.ds(rm, 8), :], blk_m.at[buf, c], sem_m)
            return 0

        lax.fori_loop(0, CHUNK // LANES, wave, 0)

    def drain(buf, sem_u, sem_m):
        def one(c, _):
            pltpu.make_async_copy(
                table_hbm.at[pl.ds(0, 8), :], blk_u.at[buf, c], sem_u).wait()
            pltpu.make_async_copy(
                table_hbm.at[pl.ds(0, 8), :], blk_m.at[buf, c], sem_m).wait()
            return 0

        lax.fori_loop(0, CHUNK, one, 0)

    def consume(k, buf):
        def mul16(c16, _):
            vu = idx_u[pl.ds(k * CHUNK + c16 * LANES, LANES)]
            vm = idx_m[pl.ds(k * CHUNK + c16 * LANES, LANES)]
            for l in range(LANES):
                b = c16 * LANES + l
                su = jnp.bitwise_and(vu[l], 7)
                sm = jnp.bitwise_and(vm[l], 7)
                v = b // 2
                p = b % 2
                for g in range(DIMS // LANES):
                    sl = pl.ds(g * LANES, LANES)
                    outb[v, pl.ds(p * DIMS + g * LANES, LANES)] = (
                        blk_u[buf, b, su, sl] * blk_m[buf, b, sm, sl])
            return 0

        lax.fori_loop(0, CHUNK // LANES, mul16, 0)

        pltpu.sync_copy(
            outb,
            out_hbm.at[pl.ds(pl.multiple_of((base + k * CHUNK) // 2, CHUNK // 2),
                             CHUNK // 2)])

    issue(0, 0, sem_u0, sem_m0)

    def chunk_pair(kk, _):
        k0 = kk * 2
        issue(k0 + 1, 1, sem_u1, sem_m1)
        drain(0, sem_u0, sem_m0)
        consume(k0, 0)

        @pl.when(k0 + 2 < N_CHUNKS)
        def _():
            issue(k0 + 2, 0, sem_u0, sem_m0)

        drain(1, sem_u1, sem_m1)
        consume(k0 + 1, 1)
        return 0

    lax.fori_loop(0, N_CHUNKS // 2, chunk_pair, 0)


@jax.jit
def _gmf(u, m, user_table):
    kfn = pl.kernel(
        _gmf_body,
        out_type=jax.ShapeDtypeStruct((BATCH // 2, VROW), jnp.float32),
        mesh=plsc.VectorSubcoreMesh(core_axis_name="c", subcore_axis_name="s"),
        scratch_types=[
            pltpu.VMEM((B_PER_W,), jnp.int32),
            pltpu.VMEM((B_PER_W,), jnp.int32),
            pltpu.VMEM((2, CHUNK, 8, DIMS), jnp.float32),
            pltpu.VMEM((2, CHUNK, 8, DIMS), jnp.float32),
            pltpu.VMEM((CHUNK // 2, VROW), jnp.float32),
            pltpu.SemaphoreType.DMA,
            pltpu.SemaphoreType.DMA,
            pltpu.SemaphoreType.DMA,
            pltpu.SemaphoreType.DMA,
        ],
    )
    packed = kfn(u, m, user_table)
    return packed.reshape(BATCH, DIMS)


def kernel(u, m, user_table, movie_table):
    return _gmf(u, m, user_table)


# speedup vs baseline: 1.0015x; 1.0015x over previous
import jax
import jax.numpy as jnp
from jax import lax
from jax.experimental import pallas as pl
from jax.experimental.pallas import tpu as pltpu
from jax.experimental.pallas import tpu_sc as plsc

BATCH = 16384
DIMS = 64
LANES = 16
VROW = 2 * DIMS

_info = plsc.get_sparse_core_info()
NC = _info.num_cores
NS = _info.num_subcores
NW = NC * NS

B_PER_W = BATCH // NW
CHUNK = 16
N_CHUNKS = B_PER_W // CHUNK


def _gmf_body(u_hbm, m_hbm, table_hbm, out_hbm,
              idx_u, idx_m, blk_u, blk_m, outb,
              sem_u0, sem_m0, sem_u1, sem_m1, sem_o0, sem_o1):
    wid = lax.axis_index("s") * NC + lax.axis_index("c")
    base = pl.multiple_of(wid * B_PER_W, B_PER_W)

    pltpu.sync_copy(u_hbm.at[pl.ds(base, B_PER_W)], idx_u)
    pltpu.sync_copy(m_hbm.at[pl.ds(base, B_PER_W)], idx_m)

    def issue(k, buf, sem_u, sem_m):
        def wave(c16, _):
            vu = idx_u[pl.ds(k * CHUNK + c16 * LANES, LANES)]
            vm = idx_m[pl.ds(k * CHUNK + c16 * LANES, LANES)]
            for l in range(LANES):
                c = c16 * LANES + l
                ru = pl.multiple_of(
                    lax.shift_left(lax.shift_right_logical(vu[l], 3), 3), 8)
                rm = pl.multiple_of(
                    lax.shift_left(lax.shift_right_logical(vm[l], 3), 3), 8)
                pltpu.async_copy(
                    table_hbm.at[pl.ds(ru, 8), :], blk_u.at[buf, c], sem_u)
                pltpu.async_copy(
                    table_hbm.at[pl.ds(rm, 8), :], blk_m.at[buf, c], sem_m)
            return 0

        lax.fori_loop(0, CHUNK // LANES, wave, 0)

    def drain(buf, sem_u, sem_m):
        def one(c, _):
            pltpu.make_async_copy(
                table_hbm.at[pl.ds(0, 8), :], blk_u.at[buf, c], sem_u).wait()
            pltpu.make_async_copy(
                table_hbm.at[pl.ds(0, 8), :], blk_m.at[buf, c], sem_m).wait()
            return 0

        lax.fori_loop(0, CHUNK, one, 0)

    def wait_out(buf, sem_o):
        pltpu.make_async_copy(
            outb.at[buf],
            out_hbm.at[pl.ds(0, CHUNK // 2)], sem_o).wait()

    def consume(k, buf, sem_o):
        def mul16(c16, _):
            vu = idx_u[pl.ds(k * CHUNK + c16 * LANES, LANES)]
            vm = idx_m[pl.ds(k * CHUNK + c16 * LANES, LANES)]
            for l in range(LANES):
                b = c16 * LANES + l
                su = jnp.bitwise_and(vu[l], 7)
                sm = jnp.bitwise_and(vm[l], 7)
                v = b // 2
                p = b % 2
                for g in range(DIMS // LANES):
                    sl = pl.ds(g * LANES, LANES)
                    outb[buf, v, pl.ds(p * DIMS + g * LANES, LANES)] = (
                        blk_u[buf, b, su, sl] * blk_m[buf, b, sm, sl])
            return 0

        lax.fori_loop(0, CHUNK // LANES, mul16, 0)

        pltpu.async_copy(
            outb.at[buf],
            out_hbm.at[pl.ds(pl.multiple_of((base + k * CHUNK) // 2, CHUNK // 2),
                             CHUNK // 2)],
            sem_o)

    issue(0, 0, sem_u0, sem_m0)

    def chunk_pair(kk, _):
        k0 = kk * 2
        issue(k0 + 1, 1, sem_u1, sem_m1)
        drain(0, sem_u0, sem_m0)

        @pl.when(k0 >= 2)
        def _():
            wait_out(0, sem_o0)

        consume(k0, 0, sem_o0)

        @pl.when(k0 + 2 < N_CHUNKS)
        def _():
            issue(k0 + 2, 0, sem_u0, sem_m0)

        drain(1, sem_u1, sem_m1)

        @pl.when(k0 >= 2)
        def _():
            wait_out(1, sem_o1)

        consume(k0 + 1, 1, sem_o1)
        return 0

    lax.fori_loop(0, N_CHUNKS // 2, chunk_pair, 0)
    wait_out(0, sem_o0)
    wait_out(1, sem_o1)


@jax.jit
def _gmf(u, m, user_table):
    kfn = pl.kernel(
        _gmf_body,
        out_type=jax.ShapeDtypeStruct((BATCH // 2, VROW), jnp.float32),
        mesh=plsc.VectorSubcoreMesh(core_axis_name="c", subcore_axis_name="s"),
        scratch_types=[
            pltpu.VMEM((B_PER_W,), jnp.int32),
            pltpu.VMEM((B_PER_W,), jnp.int32),
            pltpu.VMEM((2, CHUNK, 8, DIMS), jnp.float32),
            pltpu.VMEM((2, CHUNK, 8, DIMS), jnp.float32),
            pltpu.VMEM((2, CHUNK // 2, VROW), jnp.float32),
            pltpu.SemaphoreType.DMA,
            pltpu.SemaphoreType.DMA,
            pltpu.SemaphoreType.DMA,
            pltpu.SemaphoreType.DMA,
            pltpu.SemaphoreType.DMA,
            pltpu.SemaphoreType.DMA,
        ],
    )
    packed = kfn(u, m, user_table)
    return packed.reshape(BATCH, DIMS)


def kernel(u, m, user_table, movie_table):
    return _gmf(u, m, user_table)
